# Initial kernel scaffold; baseline (speedup 1.0000x reference)
import functools

import jax
import jax.numpy as jnp
from jax import lax
from jax.experimental import pallas as pl
from jax.experimental.pallas import tpu as pltpu
from jax.experimental.pallas import tpu_sc as plsc

NC = 2
NS = 16
NW = NC * NS
CHUNK = 128
PCOLS = 16
ZROWS = 1024


def _round_up(a, b):
    return (a + b - 1) // b * b


def _make_sc_segsum(n_nodes, n_chunks, n_pass, with_counts):
    R = _round_up(2 * n_nodes + 8, NS * 8)
    stripe = R // NS

    zsizes = []
    rem = stripe
    while rem:
        t = min(ZROWS, rem)
        zsizes.append(t)
        rem -= t

    out_type = [jax.ShapeDtypeStruct((NC, n_pass, R, PCOLS), jnp.float32)]
    if with_counts:
        out_type.append(jax.ShapeDtypeStruct((NC, R, PCOLS), jnp.float32))

    mesh = plsc.VectorSubcoreMesh(core_axis_name="c", subcore_axis_name="s")

    scratch = [
        pltpu.VMEM((n_chunks, CHUNK), jnp.int32),
        pltpu.VMEM((n_chunks, CHUNK), jnp.int32),
        pltpu.VMEM((CHUNK, PCOLS), jnp.float32),
        pltpu.VMEM((ZROWS, PCOLS), jnp.float32),
        pltpu.VMEM_SHARED((R, PCOLS), jnp.float32),
        pltpu.SemaphoreType.DMA,
    ]

    @functools.partial(pl.kernel, out_type=out_type, mesh=mesh,
                       scratch_types=scratch)
    def kern(*refs):
        tabs = refs[:n_pass]
        src_hbm, scat_hbm = refs[n_pass], refs[n_pass + 1]
        sums_hbm = refs[n_pass + 2]
        k = n_pass + 3
        cnts_hbm = refs[k] if with_counts else None
        k += 1 if with_counts else 0
        src3, scat3, rows, zbuf, acc, sem = refs[k:k + 6]

        c = lax.axis_index("c")
        s = lax.axis_index("s")
        wid = c * NS + s
        base = s * stripe

        pltpu.sync_copy(src_hbm.at[wid], src3)
        pltpu.sync_copy(scat_hbm.at[wid], scat3)

        def _zb(i, carry):
            zbuf[i] = jnp.zeros((PCOLS,), jnp.float32)
            return carry
        lax.fori_loop(0, ZROWS, _zb, 0)

        def _zero_stripe():
            off = 0
            for sz in zsizes:
                pltpu.sync_copy(zbuf.at[pl.ds(0, sz)],
                                acc.at[pl.ds(base + off, sz)])
                off += sz

        for p in range(n_pass):
            _zero_stripe()
            plsc.subcore_barrier()

            def _chunk(j, carry):
                pltpu.async_copy(tabs[p].at[src3.at[j]], rows, sem).wait()
                pltpu.sync_copy(rows, acc.at[scat3.at[j]], add=True)
                return carry
            lax.fori_loop(0, n_chunks, _chunk, 0)
            plsc.subcore_barrier()
            pltpu.sync_copy(acc.at[pl.ds(base, stripe)],
                            sums_hbm.at[c, p, pl.ds(base, stripe)])

        if with_counts:
            def _ob(i, carry):
                rows[i] = jnp.ones((PCOLS,), jnp.float32)
                return carry
            lax.fori_loop(0, CHUNK, _ob, 0)
            _zero_stripe()
            plsc.subcore_barrier()

            def _cchunk(j, carry):
                pltpu.sync_copy(rows, acc.at[scat3.at[j]], add=True)
                return carry
            lax.fori_loop(0, n_chunks, _cchunk, 0)
            plsc.subcore_barrier()
            pltpu.sync_copy(acc.at[pl.ds(base, stripe)],
                            cnts_hbm.at[c, pl.ds(base, stripe)])

    return kern, R


def _sum_cores_concat(t):
    ts = t[0] + t[1]
    return jnp.concatenate([ts[p] for p in range(ts.shape[0])], axis=-1)


def _mean(sums_blk, cnt_blk):
    cnt = (cnt_blk[0] + cnt_blk[1])[:, 0:1]
    return _sum_cores_concat(sums_blk) / jnp.maximum(cnt, 1.0)


def _dot(a, b):
    return jnp.dot(a, b, preferred_element_type=jnp.float32)


def _layer1_body(xr, spr, snr, cpr, cnr, wb, bb, wu, bu, z1r):
    x = xr[...]
    mp = _mean(spr[...], cpr[...])
    mn = _mean(snr[...], cnr[...])
    hb = _dot(mp, wb[0:64]) + _dot(x, wb[64:128]) + bb[...]
    hu = _dot(mn, wu[0:64]) + _dot(x, wu[64:128]) + bu[...]
    z1r[...] = jnp.tanh(jnp.concatenate([hb, hu], axis=1))


def _layer2_body(z1r, spr, snr, cpr, cnr, wb, bb, wu, bu, ww, bw, m1, c1,
                 zr, h1pr, statsr):
    z1 = z1r[...]
    mp = _mean(spr[...], cpr[...])
    mn = _mean(snr[...], cnr[...])
    hb = (_dot(mp[:, 0:32], wb[0:32]) + _dot(mn[:, 32:64], wb[32:64])
          + _dot(z1[:, 0:32], wb[64:96]) + bb[...])
    hu = (_dot(mp[:, 32:64], wu[0:32]) + _dot(mn[:, 0:32], wu[32:64])
          + _dot(z1[:, 32:64], wu[64:96]) + bu[...])
    z2 = jnp.tanh(jnp.concatenate([hb, hu], axis=1))
    z = jnp.tanh(_dot(z2, ww[...]) + bw[...])
    zr[...] = z
    h1p = _dot(z, m1[...]) + c1[...]
    h1pr[...] = h1p

    @pl.when(pl.program_id(0) == 0)
    def _():
        statsr[...] = jnp.zeros_like(statsr)
    statsr[0:1, :] += jnp.sum(h1p, axis=0, keepdims=True)
    statsr[1:2, :] += jnp.sum(h1p * h1p, axis=0, keepdims=True)


def _bn_relu(h, stats, g, b, n):
    m = stats[0:1, :] / n
    v = stats[1:2, :] / n - m * m
    return jnp.maximum((h - m) * lax.rsqrt(v + 1e-5) * g + b, 0.0)


def _mlp_mid_body(n_nodes, h1pr, statsr, g1, be1, m2, c2, h2pr, stats2r):
    h1 = _bn_relu(h1pr[...], statsr[...], g1[...], be1[...], n_nodes)
    h2p = _dot(h1, m2[...]) + c2[...]
    h2pr[...] = h2p

    @pl.when(pl.program_id(0) == 0)
    def _():
        stats2r[...] = jnp.zeros_like(stats2r)
    stats2r[0:1, :] += jnp.sum(h2p, axis=0, keepdims=True)
    stats2r[1:2, :] += jnp.sum(h2p * h2p, axis=0, keepdims=True)


def _mlp_out_body(n_nodes, h2pr, statsr, g2, be2, m3t, c3, probr):
    h2 = _bn_relu(h2pr[...], statsr[...], g2[...], be2[...], n_nodes)
    logit = jnp.sum(h2 * m3t[...], axis=1, keepdims=True) + c3[...]
    probr[...] = jax.nn.sigmoid(logit)


def kernel(init_emb, edge_index_s, Wb1, bb1, Wu1, bu1, Wb2, bb2, Wu2, bu2,
           Ww, bw, M1, c1, g1, be1, M2, c2, g2, be2, M3, c3):
    N, D = init_emb.shape
    E = edge_index_s.shape[0]
    H = Wb1.shape[1]
    n_pass = D // PCOLS
    n_chunks = _round_up(E, NW * CHUNK) // (NW * CHUNK)
    Ep = NW * n_chunks * CHUNK

    src = edge_index_s[:, 0].astype(jnp.int32)
    dst = edge_index_s[:, 1].astype(jnp.int32)
    sgn = edge_index_s[:, 2]
    scat = dst + N * (sgn < 0).astype(jnp.int32)
    src_r = jnp.pad(src, (0, Ep - E)).reshape(NW, n_chunks, CHUNK)
    scat_r = jnp.pad(scat, (0, Ep - E),
                     constant_values=2 * N).reshape(NW, n_chunks, CHUNK)

    sc1, R = _make_sc_segsum(N, n_chunks, n_pass, with_counts=True)
    sc2, _ = _make_sc_segsum(N, n_chunks, n_pass, with_counts=False)

    xcols = [init_emb[:, p * PCOLS:(p + 1) * PCOLS] for p in range(n_pass)]
    sums1, cnts = sc1(*xcols, src_r, scat_r)

    B = 2000
    NB = N // B
    grid = (NB,)
    f32 = jnp.float32

    spec_x = pl.BlockSpec((B, D), lambda i: (i, 0))
    spec_sp = pl.BlockSpec((NC, n_pass, B, PCOLS), lambda i: (0, 0, i, 0))
    spec_sn = pl.BlockSpec((NC, n_pass, B, PCOLS), lambda i: (0, 0, i + NB, 0))
    spec_cp = pl.BlockSpec((NC, B, PCOLS), lambda i: (0, i, 0))
    spec_cn = pl.BlockSpec((NC, B, PCOLS), lambda i: (0, i + NB, 0))

    def whole(shape):
        nd = len(shape)
        return pl.BlockSpec(shape, lambda i, _n=nd: (0,) * _n)

    z1 = pl.pallas_call(
        _layer1_body,
        grid=grid,
        in_specs=[spec_x, spec_sp, spec_sn, spec_cp, spec_cn,
                  whole((2 * D, H)), whole((1, H)),
                  whole((2 * D, H)), whole((1, H))],
        out_specs=pl.BlockSpec((B, D), lambda i: (i, 0)),
        out_shape=jax.ShapeDtypeStruct((N, D), f32),
    )(init_emb, sums1, sums1, cnts, cnts, Wb1, bb1.reshape(1, H),
      Wu1, bu1.reshape(1, H))

    zcols = [z1[:, p * PCOLS:(p + 1) * PCOLS] for p in range(n_pass)]
    (sums2,) = sc2(*zcols, src_r, scat_r)

    z, h1p, stats1 = pl.pallas_call(
        _layer2_body,
        grid=grid,
        in_specs=[spec_x, spec_sp, spec_sn, spec_cp, spec_cn,
                  whole((3 * H, H)), whole((1, H)),
                  whole((3 * H, H)), whole((1, H)),
                  whole((D, D)), whole((1, D)),
                  whole((D, D)), whole((1, D))],
        out_specs=[pl.BlockSpec((B, D), lambda i: (i, 0)),
                   pl.BlockSpec((B, D), lambda i: (i, 0)),
                   pl.BlockSpec((2, D), lambda i: (0, 0))],
        out_shape=[jax.ShapeDtypeStruct((N, D), f32),
                   jax.ShapeDtypeStruct((N, D), f32),
                   jax.ShapeDtypeStruct((2, D), f32)],
    )(z1, sums2, sums2, cnts, cnts, Wb2, bb2.reshape(1, H),
      Wu2, bu2.reshape(1, H), Ww, bw.reshape(1, D), M1, c1.reshape(1, D))

    h2p, stats2 = pl.pallas_call(
        functools.partial(_mlp_mid_body, float(N)),
        grid=grid,
        in_specs=[spec_x, whole((2, D)), whole((1, D)), whole((1, D)),
                  whole((D, D)), whole((1, D))],
        out_specs=[pl.BlockSpec((B, D), lambda i: (i, 0)),
                   pl.BlockSpec((2, D), lambda i: (0, 0))],
        out_shape=[jax.ShapeDtypeStruct((N, D), f32),
                   jax.ShapeDtypeStruct((2, D), f32)],
    )(h1p, stats1, g1.reshape(1, D), be1.reshape(1, D), M2, c2.reshape(1, D))

    prob = pl.pallas_call(
        functools.partial(_mlp_out_body, float(N)),
        grid=grid,
        in_specs=[spec_x, whole((2, D)), whole((1, D)), whole((1, D)),
                  whole((1, D)), whole((1, 1))],
        out_specs=pl.BlockSpec((B, 1), lambda i: (i, 0)),
        out_shape=jax.ShapeDtypeStruct((N, 1), f32),
    )(h2p, stats2, g2.reshape(1, D), be2.reshape(1, D),
      M3.reshape(1, D), c3.reshape(1, 1))

    return (z, prob)


# speedup vs baseline: 3.9326x; 3.9326x over previous
import functools

import jax
import jax.numpy as jnp
from jax import lax
from jax.experimental import pallas as pl
from jax.experimental.pallas import tpu as pltpu
from jax.experimental.pallas import tpu_sc as plsc

NC = 2
NS = 16
NW = NC * NS
CHUNK = 128
GC = 16
PCOLS = 16
ZROWS = 1024


def _round_up(a, b):
    return (a + b - 1) // b * b


def _make_sc_segsum(n_nodes, n_groups, n_pass, with_counts):
    R = _round_up(2 * n_nodes + 8, NS * 8)
    stripe = R // NS

    zsizes = []
    rem = stripe
    while rem:
        t = min(ZROWS, rem)
        zsizes.append(t)
        rem -= t

    out_type = [jax.ShapeDtypeStruct((NC, n_pass, R, PCOLS), jnp.float32)]
    if with_counts:
        out_type.append(jax.ShapeDtypeStruct((NC, R, PCOLS), jnp.float32))

    mesh = plsc.VectorSubcoreMesh(core_axis_name="c", subcore_axis_name="s")

    scratch = [
        pltpu.VMEM((GC, CHUNK), jnp.int32),
        pltpu.VMEM((GC, CHUNK), jnp.int32),
        pltpu.VMEM((CHUNK, PCOLS), jnp.float32),
        pltpu.VMEM((ZROWS, PCOLS), jnp.float32),
        pltpu.VMEM_SHARED((R, PCOLS), jnp.float32),
        pltpu.SemaphoreType.DMA,
    ]

    @functools.partial(pl.kernel, out_type=out_type, mesh=mesh,
                       scratch_types=scratch,
                       compiler_params=pltpu.CompilerParams(
                           use_tc_tiling_on_sc=False))
    def kern(*refs):
        tabs = refs[:n_pass]
        src_hbm, scat_hbm = refs[n_pass], refs[n_pass + 1]
        sums_hbm = refs[n_pass + 2]
        k = n_pass + 3
        cnts_hbm = refs[k] if with_counts else None
        k += 1 if with_counts else 0
        srcb, scatb, rows, zbuf, acc, sem = refs[k:k + 6]

        c = lax.axis_index("c")
        s = lax.axis_index("s")
        wid = c * NS + s
        base = s * stripe

        def _zb(i, carry):
            zbuf[i] = jnp.zeros((PCOLS,), jnp.float32)
            return carry
        lax.fori_loop(0, ZROWS, _zb, 0)

        def _zero_stripe():
            off = 0
            for sz in zsizes:
                pltpu.sync_copy(zbuf.at[pl.ds(0, sz)],
                                acc.at[pl.ds(base + off, sz)])
                off += sz

        for p in range(n_pass):
            _zero_stripe()
            plsc.subcore_barrier()

            def _group(g, carry):
                pltpu.sync_copy(src_hbm.at[wid, g], srcb)
                pltpu.sync_copy(scat_hbm.at[wid, g], scatb)

                def _chunk(j, carry2):
                    pltpu.async_copy(tabs[p].at[srcb.at[j]], rows, sem).wait()
                    pltpu.sync_copy(rows, acc.at[scatb.at[j]], add=True)
                    return carry2
                return lax.fori_loop(0, GC, _chunk, carry)
            lax.fori_loop(0, n_groups, _group, 0)
            plsc.subcore_barrier()
            pltpu.sync_copy(acc.at[pl.ds(base, stripe)],
                            sums_hbm.at[c, p, pl.ds(base, stripe)])

        if with_counts:
            def _ob(i, carry):
                rows[i] = jnp.ones((PCOLS,), jnp.float32)
                return carry
            lax.fori_loop(0, CHUNK, _ob, 0)
            _zero_stripe()
            plsc.subcore_barrier()

            def _cgroup(g, carry):
                pltpu.sync_copy(scat_hbm.at[wid, g], scatb)

                def _cchunk(j, carry2):
                    pltpu.sync_copy(rows, acc.at[scatb.at[j]], add=True)
                    return carry2
                return lax.fori_loop(0, GC, _cchunk, carry)
            lax.fori_loop(0, n_groups, _cgroup, 0)
            plsc.subcore_barrier()
            pltpu.sync_copy(acc.at[pl.ds(base, stripe)],
                            cnts_hbm.at[c, pl.ds(base, stripe)])

    return kern, R


def _sum_cores_concat(t):
    ts = t[0] + t[1]
    return jnp.concatenate([ts[p] for p in range(ts.shape[0])], axis=-1)


def _mean(sums_blk, cnt_blk):
    cnt = (cnt_blk[0] + cnt_blk[1])[:, 0:1]
    return _sum_cores_concat(sums_blk) / jnp.maximum(cnt, 1.0)


def _dot(a, b):
    return jnp.dot(a, b, preferred_element_type=jnp.float32)


def _layer1_body(xr, spr, snr, cpr, cnr, wb, bb, wu, bu, z1r):
    x = xr[...]
    mp = _mean(spr[...], cpr[...])
    mn = _mean(snr[...], cnr[...])
    hb = _dot(mp, wb[0:64]) + _dot(x, wb[64:128]) + bb[...]
    hu = _dot(mn, wu[0:64]) + _dot(x, wu[64:128]) + bu[...]
    z1r[...] = jnp.tanh(jnp.concatenate([hb, hu], axis=1))


def _layer2_body(z1r, spr, snr, cpr, cnr, wb, bb, wu, bu, ww, bw, m1, c1,
                 zr, h1pr, statsr):
    z1 = z1r[...]
    mp = _mean(spr[...], cpr[...])
    mn = _mean(snr[...], cnr[...])
    hb = (_dot(mp[:, 0:32], wb[0:32]) + _dot(mn[:, 32:64], wb[32:64])
          + _dot(z1[:, 0:32], wb[64:96]) + bb[...])
    hu = (_dot(mp[:, 32:64], wu[0:32]) + _dot(mn[:, 0:32], wu[32:64])
          + _dot(z1[:, 32:64], wu[64:96]) + bu[...])
    z2 = jnp.tanh(jnp.concatenate([hb, hu], axis=1))
    z = jnp.tanh(_dot(z2, ww[...]) + bw[...])
    zr[...] = z
    h1p = _dot(z, m1[...]) + c1[...]
    h1pr[...] = h1p

    @pl.when(pl.program_id(0) == 0)
    def _():
        statsr[...] = jnp.zeros_like(statsr)
    statsr[0:1, :] += jnp.sum(h1p, axis=0, keepdims=True)
    statsr[1:2, :] += jnp.sum(h1p * h1p, axis=0, keepdims=True)


def _bn_relu(h, stats, g, b, n):
    m = stats[0:1, :] / n
    v = stats[1:2, :] / n - m * m
    return jnp.maximum((h - m) * lax.rsqrt(v + 1e-5) * g + b, 0.0)


def _mlp_mid_body(n_nodes, h1pr, statsr, g1, be1, m2, c2, h2pr, stats2r):
    h1 = _bn_relu(h1pr[...], statsr[...], g1[...], be1[...], n_nodes)
    h2p = _dot(h1, m2[...]) + c2[...]
    h2pr[...] = h2p

    @pl.when(pl.program_id(0) == 0)
    def _():
        stats2r[...] = jnp.zeros_like(stats2r)
    stats2r[0:1, :] += jnp.sum(h2p, axis=0, keepdims=True)
    stats2r[1:2, :] += jnp.sum(h2p * h2p, axis=0, keepdims=True)


def _mlp_out_body(n_nodes, h2pr, statsr, g2, be2, m3t, c3, probr):
    h2 = _bn_relu(h2pr[...], statsr[...], g2[...], be2[...], n_nodes)
    logit = jnp.sum(h2 * m3t[...], axis=1, keepdims=True) + c3[...]
    probr[...] = jax.nn.sigmoid(logit)


def kernel(init_emb, edge_index_s, Wb1, bb1, Wu1, bu1, Wb2, bb2, Wu2, bu2,
           Ww, bw, M1, c1, g1, be1, M2, c2, g2, be2, M3, c3):
    N, D = init_emb.shape
    E = edge_index_s.shape[0]
    H = Wb1.shape[1]
    n_pass = D // PCOLS
    n_groups = _round_up(E, NW * GC * CHUNK) // (NW * GC * CHUNK)
    Ep = NW * n_groups * GC * CHUNK

    src = edge_index_s[:, 0].astype(jnp.int32)
    dst = edge_index_s[:, 1].astype(jnp.int32)
    sgn = edge_index_s[:, 2]
    scat = dst + N * (sgn < 0).astype(jnp.int32)
    src_r = jnp.pad(src, (0, Ep - E)).reshape(NW, n_groups, GC, CHUNK)
    scat_r = jnp.pad(scat, (0, Ep - E),
                     constant_values=2 * N).reshape(NW, n_groups, GC, CHUNK)

    sc1, R = _make_sc_segsum(N, n_groups, n_pass, with_counts=True)
    sc2, _ = _make_sc_segsum(N, n_groups, n_pass, with_counts=False)

    xcols = [init_emb[:, p * PCOLS:(p + 1) * PCOLS] for p in range(n_pass)]
    sums1, cnts = sc1(*xcols, src_r, scat_r)

    B = 2000
    NB = N // B
    grid = (NB,)
    f32 = jnp.float32

    spec_x = pl.BlockSpec((B, D), lambda i: (i, 0))
    spec_sp = pl.BlockSpec((NC, n_pass, B, PCOLS), lambda i: (0, 0, i, 0))
    spec_sn = pl.BlockSpec((NC, n_pass, B, PCOLS), lambda i: (0, 0, i + NB, 0))
    spec_cp = pl.BlockSpec((NC, B, PCOLS), lambda i: (0, i, 0))
    spec_cn = pl.BlockSpec((NC, B, PCOLS), lambda i: (0, i + NB, 0))

    def whole(shape):
        nd = len(shape)
        return pl.BlockSpec(shape, lambda i, _n=nd: (0,) * _n)

    z1 = pl.pallas_call(
        _layer1_body,
        grid=grid,
        in_specs=[spec_x, spec_sp, spec_sn, spec_cp, spec_cn,
                  whole((2 * D, H)), whole((1, H)),
                  whole((2 * D, H)), whole((1, H))],
        out_specs=pl.BlockSpec((B, D), lambda i: (i, 0)),
        out_shape=jax.ShapeDtypeStruct((N, D), f32),
    )(init_emb, sums1, sums1, cnts, cnts, Wb1, bb1.reshape(1, H),
      Wu1, bu1.reshape(1, H))

    zcols = [z1[:, p * PCOLS:(p + 1) * PCOLS] for p in range(n_pass)]
    (sums2,) = sc2(*zcols, src_r, scat_r)

    z, h1p, stats1 = pl.pallas_call(
        _layer2_body,
        grid=grid,
        in_specs=[spec_x, spec_sp, spec_sn, spec_cp, spec_cn,
                  whole((3 * H, H)), whole((1, H)),
                  whole((3 * H, H)), whole((1, H)),
                  whole((D, D)), whole((1, D)),
                  whole((D, D)), whole((1, D))],
        out_specs=[pl.BlockSpec((B, D), lambda i: (i, 0)),
                   pl.BlockSpec((B, D), lambda i: (i, 0)),
                   pl.BlockSpec((2, D), lambda i: (0, 0))],
        out_shape=[jax.ShapeDtypeStruct((N, D), f32),
                   jax.ShapeDtypeStruct((N, D), f32),
                   jax.ShapeDtypeStruct((2, D), f32)],
    )(z1, sums2, sums2, cnts, cnts, Wb2, bb2.reshape(1, H),
      Wu2, bu2.reshape(1, H), Ww, bw.reshape(1, D), M1, c1.reshape(1, D))

    h2p, stats2 = pl.pallas_call(
        functools.partial(_mlp_mid_body, float(N)),
        grid=grid,
        in_specs=[spec_x, whole((2, D)), whole((1, D)), whole((1, D)),
                  whole((D, D)), whole((1, D))],
        out_specs=[pl.BlockSpec((B, D), lambda i: (i, 0)),
                   pl.BlockSpec((2, D), lambda i: (0, 0))],
        out_shape=[jax.ShapeDtypeStruct((N, D), f32),
                   jax.ShapeDtypeStruct((2, D), f32)],
    )(h1p, stats1, g1.reshape(1, D), be1.reshape(1, D), M2, c2.reshape(1, D))

    prob = pl.pallas_call(
        functools.partial(_mlp_out_body, float(N)),
        grid=grid,
        in_specs=[spec_x, whole((2, D)), whole((1, D)), whole((1, D)),
                  whole((1, D)), whole((1, 1))],
        out_specs=pl.BlockSpec((B, 1), lambda i: (i, 0)),
        out_shape=jax.ShapeDtypeStruct((N, 1), f32),
    )(h2p, stats2, g2.reshape(1, D), be2.reshape(1, D),
      M3.reshape(1, D), c3.reshape(1, 1))

    return (z, prob)


# speedup vs baseline: 5.8758x; 1.4941x over previous
import functools

import jax
import jax.numpy as jnp
from jax import lax
from jax.experimental import pallas as pl
from jax.experimental.pallas import tpu as pltpu
from jax.experimental.pallas import tpu_sc as plsc

NC = 2
NS = 16
NW = NC * NS
CHUNK = 128
GC = 8
NBUF = 8
PCOLS = 16


def _round_up(a, b):
    return (a + b - 1) // b * b


def _make_sc_segsum(n_nodes, n_groups, n_pass, with_counts):
    R = _round_up(2 * n_nodes + 8, NS * 8)
    stripe = R // NS
    nzfull = stripe // CHUNK
    ztail = stripe - nzfull * CHUNK

    out_type = [jax.ShapeDtypeStruct((NC, n_pass, R, PCOLS), jnp.float32)]
    if with_counts:
        out_type.append(jax.ShapeDtypeStruct((NC, R, PCOLS), jnp.float32))

    mesh = plsc.VectorSubcoreMesh(core_axis_name="c", subcore_axis_name="s")

    scratch = (
        [pltpu.VMEM((GC, CHUNK), jnp.int32),
         pltpu.VMEM((GC, CHUNK), jnp.int32)]
        + [pltpu.VMEM((CHUNK, PCOLS), jnp.float32) for _ in range(NBUF)]
        + [pltpu.VMEM_SHARED((R, PCOLS), jnp.float32),
           pltpu.SemaphoreType.DMA,
           pltpu.SemaphoreType.DMA]
    )

    @functools.partial(pl.kernel, out_type=out_type, mesh=mesh,
                       scratch_types=scratch,
                       compiler_params=pltpu.CompilerParams(
                           use_tc_tiling_on_sc=False))
    def kern(*refs):
        tabs = refs[:n_pass]
        src_hbm, scat_hbm = refs[n_pass], refs[n_pass + 1]
        sums_hbm = refs[n_pass + 2]
        k = n_pass + 3
        cnts_hbm = refs[k] if with_counts else None
        k += 1 if with_counts else 0
        srcb, scatb = refs[k], refs[k + 1]
        rows = refs[k + 2:k + 2 + NBUF]
        acc, gsem, ssem = refs[k + 2 + NBUF:k + 5 + NBUF]

        c = lax.axis_index("c")
        s = lax.axis_index("s")
        wid = c * NS + s
        base = s * stripe

        def _fill(buf, val):
            def f(i, carry):
                buf[i] = jnp.full((PCOLS,), val, jnp.float32)
                return carry
            lax.fori_loop(0, CHUNK, f, 0)

        def _zero_stripe():
            def zf(i, carry):
                pltpu.async_copy(
                    rows[0], acc.at[pl.ds(base + i * CHUNK, CHUNK)], ssem)
                return carry
            lax.fori_loop(0, nzfull, zf, 0)

            def zw(i, carry):
                pltpu.make_async_copy(
                    rows[0], acc.at[pl.ds(base + i * CHUNK, CHUNK)],
                    ssem).wait()
                return carry
            lax.fori_loop(0, nzfull, zw, 0)
            if ztail:
                pltpu.sync_copy(
                    rows[0].at[pl.ds(0, ztail)],
                    acc.at[pl.ds(base + nzfull * CHUNK, ztail)])

        H1 = GC // 2

        for p in range(n_pass):
            _fill(rows[0], 0.0)
            _zero_stripe()
            plsc.subcore_barrier()

            def _group(g, carry):
                pltpu.sync_copy(src_hbm.at[wid, g], srcb)
                pltpu.sync_copy(scat_hbm.at[wid, g], scatb)
                gd = [pltpu.async_copy(tabs[p].at[srcb.at[j]], rows[j], gsem)
                      for j in range(GC)]
                for j in range(H1):
                    gd[j].wait()
                sd = [pltpu.async_copy(rows[j], acc.at[scatb.at[j]], ssem,
                                       add=True)
                      for j in range(H1)]
                for j in range(H1, GC):
                    gd[j].wait()
                for d in sd:
                    d.wait()
                sd2 = [pltpu.async_copy(rows[j], acc.at[scatb.at[j]], ssem,
                                        add=True)
                       for j in range(H1, GC)]
                for d in sd2:
                    d.wait()
                return carry
            lax.fori_loop(0, n_groups, _group, 0)
            plsc.subcore_barrier()
            pltpu.sync_copy(acc.at[pl.ds(base, stripe)],
                            sums_hbm.at[c, p, pl.ds(base, stripe)])

        if with_counts:
            _fill(rows[0], 0.0)
            _zero_stripe()
            plsc.subcore_barrier()
            for b in range(NBUF):
                _fill(rows[b], 1.0)

            def _cgroup(g, carry):
                pltpu.sync_copy(scat_hbm.at[wid, g], scatb)
                sd = [pltpu.async_copy(rows[j % NBUF], acc.at[scatb.at[j]],
                                       ssem, add=True)
                      for j in range(GC)]
                for d in sd:
                    d.wait()
                return carry
            lax.fori_loop(0, n_groups, _cgroup, 0)
            plsc.subcore_barrier()
            pltpu.sync_copy(acc.at[pl.ds(base, stripe)],
                            cnts_hbm.at[c, pl.ds(base, stripe)])

    return kern, R


def _sum_cores_concat(t):
    ts = t[0] + t[1]
    return jnp.concatenate([ts[p] for p in range(ts.shape[0])], axis=-1)


def _mean(sums_blk, cnt_blk):
    cnt = (cnt_blk[0] + cnt_blk[1])[:, 0:1]
    return _sum_cores_concat(sums_blk) / jnp.maximum(cnt, 1.0)


def _dot(a, b):
    return jnp.dot(a, b, preferred_element_type=jnp.float32)


def _layer1_body(xr, spr, snr, cpr, cnr, wb, bb, wu, bu, z1r):
    x = xr[...]
    mp = _mean(spr[...], cpr[...])
    mn = _mean(snr[...], cnr[...])
    hb = _dot(mp, wb[0:64]) + _dot(x, wb[64:128]) + bb[...]
    hu = _dot(mn, wu[0:64]) + _dot(x, wu[64:128]) + bu[...]
    z1r[...] = jnp.tanh(jnp.concatenate([hb, hu], axis=1))


def _layer2_body(z1r, spr, snr, cpr, cnr, wb, bb, wu, bu, ww, bw, m1, c1,
                 zr, h1pr, statsr):
    z1 = z1r[...]
    mp = _mean(spr[...], cpr[...])
    mn = _mean(snr[...], cnr[...])
    hb = (_dot(mp[:, 0:32], wb[0:32]) + _dot(mn[:, 32:64], wb[32:64])
          + _dot(z1[:, 0:32], wb[64:96]) + bb[...])
    hu = (_dot(mp[:, 32:64], wu[0:32]) + _dot(mn[:, 0:32], wu[32:64])
          + _dot(z1[:, 32:64], wu[64:96]) + bu[...])
    z2 = jnp.tanh(jnp.concatenate([hb, hu], axis=1))
    z = jnp.tanh(_dot(z2, ww[...]) + bw[...])
    zr[...] = z
    h1p = _dot(z, m1[...]) + c1[...]
    h1pr[...] = h1p

    @pl.when(pl.program_id(0) == 0)
    def _():
        statsr[...] = jnp.zeros_like(statsr)
    statsr[0:1, :] += jnp.sum(h1p, axis=0, keepdims=True)
    statsr[1:2, :] += jnp.sum(h1p * h1p, axis=0, keepdims=True)


def _bn_relu(h, stats, g, b, n):
    m = stats[0:1, :] / n
    v = stats[1:2, :] / n - m * m
    return jnp.maximum((h - m) * lax.rsqrt(v + 1e-5) * g + b, 0.0)


def _mlp_mid_body(n_nodes, h1pr, statsr, g1, be1, m2, c2, h2pr, stats2r):
    h1 = _bn_relu(h1pr[...], statsr[...], g1[...], be1[...], n_nodes)
    h2p = _dot(h1, m2[...]) + c2[...]
    h2pr[...] = h2p

    @pl.when(pl.program_id(0) == 0)
    def _():
        stats2r[...] = jnp.zeros_like(stats2r)
    stats2r[0:1, :] += jnp.sum(h2p, axis=0, keepdims=True)
    stats2r[1:2, :] += jnp.sum(h2p * h2p, axis=0, keepdims=True)


def _mlp_out_body(n_nodes, h2pr, statsr, g2, be2, m3t, c3, probr):
    h2 = _bn_relu(h2pr[...], statsr[...], g2[...], be2[...], n_nodes)
    logit = jnp.sum(h2 * m3t[...], axis=1, keepdims=True) + c3[...]
    probr[...] = jax.nn.sigmoid(logit)


def kernel(init_emb, edge_index_s, Wb1, bb1, Wu1, bu1, Wb2, bb2, Wu2, bu2,
           Ww, bw, M1, c1, g1, be1, M2, c2, g2, be2, M3, c3):
    N, D = init_emb.shape
    E = edge_index_s.shape[0]
    H = Wb1.shape[1]
    n_pass = D // PCOLS
    n_groups = _round_up(E, NW * GC * CHUNK) // (NW * GC * CHUNK)
    Ep = NW * n_groups * GC * CHUNK

    src = edge_index_s[:, 0].astype(jnp.int32)
    dst = edge_index_s[:, 1].astype(jnp.int32)
    sgn = edge_index_s[:, 2]
    scat = dst + N * (sgn < 0).astype(jnp.int32)
    src_r = jnp.pad(src, (0, Ep - E)).reshape(NW, n_groups, GC, CHUNK)
    scat_r = jnp.pad(scat, (0, Ep - E),
                     constant_values=2 * N).reshape(NW, n_groups, GC, CHUNK)

    sc1, R = _make_sc_segsum(N, n_groups, n_pass, with_counts=True)
    sc2, _ = _make_sc_segsum(N, n_groups, n_pass, with_counts=False)

    xcols = [init_emb[:, p * PCOLS:(p + 1) * PCOLS] for p in range(n_pass)]
    sums1, cnts = sc1(*xcols, src_r, scat_r)

    B = 2000
    NB = N // B
    grid = (NB,)
    f32 = jnp.float32

    spec_x = pl.BlockSpec((B, D), lambda i: (i, 0))
    spec_sp = pl.BlockSpec((NC, n_pass, B, PCOLS), lambda i: (0, 0, i, 0))
    spec_sn = pl.BlockSpec((NC, n_pass, B, PCOLS), lambda i: (0, 0, i + NB, 0))
    spec_cp = pl.BlockSpec((NC, B, PCOLS), lambda i: (0, i, 0))
    spec_cn = pl.BlockSpec((NC, B, PCOLS), lambda i: (0, i + NB, 0))

    def whole(shape):
        nd = len(shape)
        return pl.BlockSpec(shape, lambda i, _n=nd: (0,) * _n)

    z1 = pl.pallas_call(
        _layer1_body,
        grid=grid,
        in_specs=[spec_x, spec_sp, spec_sn, spec_cp, spec_cn,
                  whole((2 * D, H)), whole((1, H)),
                  whole((2 * D, H)), whole((1, H))],
        out_specs=pl.BlockSpec((B, D), lambda i: (i, 0)),
        out_shape=jax.ShapeDtypeStruct((N, D), f32),
    )(init_emb, sums1, sums1, cnts, cnts, Wb1, bb1.reshape(1, H),
      Wu1, bu1.reshape(1, H))

    zcols = [z1[:, p * PCOLS:(p + 1) * PCOLS] for p in range(n_pass)]
    (sums2,) = sc2(*zcols, src_r, scat_r)

    z, h1p, stats1 = pl.pallas_call(
        _layer2_body,
        grid=grid,
        in_specs=[spec_x, spec_sp, spec_sn, spec_cp, spec_cn,
                  whole((3 * H, H)), whole((1, H)),
                  whole((3 * H, H)), whole((1, H)),
                  whole((D, D)), whole((1, D)),
                  whole((D, D)), whole((1, D))],
        out_specs=[pl.BlockSpec((B, D), lambda i: (i, 0)),
                   pl.BlockSpec((B, D), lambda i: (i, 0)),
                   pl.BlockSpec((2, D), lambda i: (0, 0))],
        out_shape=[jax.ShapeDtypeStruct((N, D), f32),
                   jax.ShapeDtypeStruct((N, D), f32),
                   jax.ShapeDtypeStruct((2, D), f32)],
    )(z1, sums2, sums2, cnts, cnts, Wb2, bb2.reshape(1, H),
      Wu2, bu2.reshape(1, H), Ww, bw.reshape(1, D), M1, c1.reshape(1, D))

    h2p, stats2 = pl.pallas_call(
        functools.partial(_mlp_mid_body, float(N)),
        grid=grid,
        in_specs=[spec_x, whole((2, D)), whole((1, D)), whole((1, D)),
                  whole((D, D)), whole((1, D))],
        out_specs=[pl.BlockSpec((B, D), lambda i: (i, 0)),
                   pl.BlockSpec((2, D), lambda i: (0, 0))],
        out_shape=[jax.ShapeDtypeStruct((N, D), f32),
                   jax.ShapeDtypeStruct((2, D), f32)],
    )(h1p, stats1, g1.reshape(1, D), be1.reshape(1, D), M2, c2.reshape(1, D))

    prob = pl.pallas_call(
        functools.partial(_mlp_out_body, float(N)),
        grid=grid,
        in_specs=[spec_x, whole((2, D)), whole((1, D)), whole((1, D)),
                  whole((1, D)), whole((1, 1))],
        out_specs=pl.BlockSpec((B, 1), lambda i: (i, 0)),
        out_shape=jax.ShapeDtypeStruct((N, 1), f32),
    )(h2p, stats2, g2.reshape(1, D), be2.reshape(1, D),
      M3.reshape(1, D), c3.reshape(1, 1))

    return (z, prob)


# speedup vs baseline: 9.1034x; 1.5493x over previous
import functools

import jax
import jax.numpy as jnp
from jax import lax
from jax.experimental import pallas as pl
from jax.experimental.pallas import tpu as pltpu
from jax.experimental.pallas import tpu_sc as plsc

NC = 2
NS = 16
NW = NC * NS
CHUNK = 128
SUB = 8
NSUB = 5
GC = SUB * NSUB
NBUF = 8
PCOLS = 16


def _round_up(a, b):
    return (a + b - 1) // b * b


def _make_sc_segsum(n_nodes, n_groups, n_pass, with_counts):
    R = _round_up(2 * n_nodes + 8, NS * 8)
    stripe = R // NS
    nzfull = stripe // CHUNK
    ztail = stripe - nzfull * CHUNK

    out_type = [jax.ShapeDtypeStruct((NC, n_pass, R, PCOLS), jnp.float32)]
    if with_counts:
        out_type.append(jax.ShapeDtypeStruct((NC, R, PCOLS), jnp.float32))

    mesh = plsc.VectorSubcoreMesh(core_axis_name="c", subcore_axis_name="s")

    scratch = (
        [pltpu.VMEM((GC, CHUNK), jnp.int32),
         pltpu.VMEM((GC, CHUNK), jnp.int32)]
        + [pltpu.VMEM((CHUNK, PCOLS), jnp.float32) for _ in range(NBUF)]
        + [pltpu.VMEM_SHARED((R, PCOLS), jnp.float32),
           pltpu.SemaphoreType.DMA,
           pltpu.SemaphoreType.DMA]
    )

    @functools.partial(pl.kernel, out_type=out_type, mesh=mesh,
                       scratch_types=scratch,
                       compiler_params=pltpu.CompilerParams(
                           use_tc_tiling_on_sc=False))
    def kern(*refs):
        tabs = refs[:n_pass]
        src_hbm, scat_hbm = refs[n_pass], refs[n_pass + 1]
        sums_hbm = refs[n_pass + 2]
        k = n_pass + 3
        cnts_hbm = refs[k] if with_counts else None
        k += 1 if with_counts else 0
        srcb, scatb = refs[k], refs[k + 1]
        rows = refs[k + 2:k + 2 + NBUF]
        acc, gsem, ssem = refs[k + 2 + NBUF:k + 5 + NBUF]

        c = lax.axis_index("c")
        s = lax.axis_index("s")
        wid = c * NS + s
        base = s * stripe

        def _fill(buf, val):
            def f(i, carry):
                buf[i] = jnp.full((PCOLS,), val, jnp.float32)
                return carry
            lax.fori_loop(0, CHUNK, f, 0)

        def _zero_stripe():
            def zf(i, carry):
                pltpu.async_copy(
                    rows[0], acc.at[pl.ds(base + i * CHUNK, CHUNK)], ssem)
                return carry
            lax.fori_loop(0, nzfull, zf, 0)

            def zw(i, carry):
                pltpu.make_async_copy(
                    rows[0], acc.at[pl.ds(base + i * CHUNK, CHUNK)],
                    ssem).wait()
                return carry
            lax.fori_loop(0, nzfull, zw, 0)
            if ztail:
                pltpu.sync_copy(
                    rows[0].at[pl.ds(0, ztail)],
                    acc.at[pl.ds(base + nzfull * CHUNK, ztail)])

        H1 = SUB // 2

        for p in range(n_pass):
            _fill(rows[0], 0.0)
            _zero_stripe()
            plsc.subcore_barrier()

            def _group(g, carry):
                pltpu.sync_copy(src_hbm.at[wid, g], srcb)
                pltpu.sync_copy(scat_hbm.at[wid, g], scatb)

                def _sub(t, carry2):
                    b0 = t * SUB
                    gd = [pltpu.async_copy(tabs[p].at[srcb.at[b0 + j]],
                                           rows[j], gsem)
                          for j in range(SUB)]
                    for j in range(H1):
                        gd[j].wait()
                    sd = [pltpu.async_copy(rows[j], acc.at[scatb.at[b0 + j]],
                                           ssem, add=True)
                          for j in range(H1)]
                    for j in range(H1, SUB):
                        gd[j].wait()
                    for d in sd:
                        d.wait()
                    sd2 = [pltpu.async_copy(rows[j], acc.at[scatb.at[b0 + j]],
                                            ssem, add=True)
                           for j in range(H1, SUB)]
                    for d in sd2:
                        d.wait()
                    return carry2
                return lax.fori_loop(0, NSUB, _sub, carry)
            lax.fori_loop(0, n_groups, _group, 0)
            plsc.subcore_barrier()
            pltpu.sync_copy(acc.at[pl.ds(base, stripe)],
                            sums_hbm.at[c, p, pl.ds(base, stripe)])

        if with_counts:
            _fill(rows[0], 0.0)
            _zero_stripe()
            plsc.subcore_barrier()
            for b in range(NBUF):
                _fill(rows[b], 1.0)

            def _cgroup(g, carry):
                pltpu.sync_copy(scat_hbm.at[wid, g], scatb)

                def _csub(t, carry2):
                    b0 = t * SUB
                    sd = [pltpu.async_copy(rows[j], acc.at[scatb.at[b0 + j]],
                                           ssem, add=True)
                          for j in range(SUB)]
                    for d in sd:
                        d.wait()
                    return carry2
                return lax.fori_loop(0, NSUB, _csub, carry)
            lax.fori_loop(0, n_groups, _cgroup, 0)
            plsc.subcore_barrier()
            pltpu.sync_copy(acc.at[pl.ds(base, stripe)],
                            cnts_hbm.at[c, pl.ds(base, stripe)])

    return kern, R


def _sum_cores_concat(t):
    ts = t[0] + t[1]
    return jnp.concatenate([ts[p] for p in range(ts.shape[0])], axis=-1)


def _mean(sums_blk, cnt_blk):
    cnt = (cnt_blk[0] + cnt_blk[1])[:, 0:1]
    return _sum_cores_concat(sums_blk) / jnp.maximum(cnt, 1.0)


def _dot(a, b):
    return jnp.dot(a, b, preferred_element_type=jnp.float32)


def _layer1_body(xr, spr, snr, cpr, cnr, wb, bb, wu, bu, z1r, *zcr):
    x = xr[...]
    mp = _mean(spr[...], cpr[...])
    mn = _mean(snr[...], cnr[...])
    hb = _dot(mp, wb[0:64]) + _dot(x, wb[64:128]) + bb[...]
    hu = _dot(mn, wu[0:64]) + _dot(x, wu[64:128]) + bu[...]
    z1 = jnp.tanh(jnp.concatenate([hb, hu], axis=1))
    z1r[...] = z1
    for p, r in enumerate(zcr):
        r[...] = z1[:, p * PCOLS:(p + 1) * PCOLS]


def _layer2_body(z1r, spr, snr, cpr, cnr, wb, bb, wu, bu, ww, bw, m1, c1,
                 zr, h1pr, statsr):
    z1 = z1r[...]
    mp = _mean(spr[...], cpr[...])
    mn = _mean(snr[...], cnr[...])
    hb = (_dot(mp[:, 0:32], wb[0:32]) + _dot(mn[:, 32:64], wb[32:64])
          + _dot(z1[:, 0:32], wb[64:96]) + bb[...])
    hu = (_dot(mp[:, 32:64], wu[0:32]) + _dot(mn[:, 0:32], wu[32:64])
          + _dot(z1[:, 32:64], wu[64:96]) + bu[...])
    z2 = jnp.tanh(jnp.concatenate([hb, hu], axis=1))
    z = jnp.tanh(_dot(z2, ww[...]) + bw[...])
    zr[...] = z
    h1p = _dot(z, m1[...]) + c1[...]
    h1pr[...] = h1p

    @pl.when(pl.program_id(0) == 0)
    def _():
        statsr[...] = jnp.zeros_like(statsr)
    statsr[0:1, :] += jnp.sum(h1p, axis=0, keepdims=True)
    statsr[1:2, :] += jnp.sum(h1p * h1p, axis=0, keepdims=True)


def _bn_relu(h, stats, g, b, n):
    m = stats[0:1, :] / n
    v = stats[1:2, :] / n - m * m
    return jnp.maximum((h - m) * lax.rsqrt(v + 1e-5) * g + b, 0.0)


def _mlp_mid_body(n_nodes, h1pr, statsr, g1, be1, m2, c2, h2pr, stats2r):
    h1 = _bn_relu(h1pr[...], statsr[...], g1[...], be1[...], n_nodes)
    h2p = _dot(h1, m2[...]) + c2[...]
    h2pr[...] = h2p

    @pl.when(pl.program_id(0) == 0)
    def _():
        stats2r[...] = jnp.zeros_like(stats2r)
    stats2r[0:1, :] += jnp.sum(h2p, axis=0, keepdims=True)
    stats2r[1:2, :] += jnp.sum(h2p * h2p, axis=0, keepdims=True)


def _mlp_out_body(n_nodes, h2pr, statsr, g2, be2, m3t, c3, probr):
    h2 = _bn_relu(h2pr[...], statsr[...], g2[...], be2[...], n_nodes)
    logit = jnp.sum(h2 * m3t[...], axis=1, keepdims=True) + c3[...]
    probr[...] = jax.nn.sigmoid(logit)


def kernel(init_emb, edge_index_s, Wb1, bb1, Wu1, bu1, Wb2, bb2, Wu2, bu2,
           Ww, bw, M1, c1, g1, be1, M2, c2, g2, be2, M3, c3):
    N, D = init_emb.shape
    E = edge_index_s.shape[0]
    H = Wb1.shape[1]
    n_pass = D // PCOLS
    n_groups = _round_up(E, NW * GC * CHUNK) // (NW * GC * CHUNK)
    Ep = NW * n_groups * GC * CHUNK

    src = edge_index_s[:, 0].astype(jnp.int32)
    dst = edge_index_s[:, 1].astype(jnp.int32)
    sgn = edge_index_s[:, 2]
    scat = dst + N * (sgn < 0).astype(jnp.int32)
    pad_ar = jnp.arange(Ep - E, dtype=jnp.int32)
    src_r = jnp.concatenate([src, pad_ar % N]).reshape(NW, n_groups, GC, CHUNK)
    scat_r = jnp.concatenate([scat, 2 * N + pad_ar % 88]).reshape(
        NW, n_groups, GC, CHUNK)

    sc1, R = _make_sc_segsum(N, n_groups, n_pass, with_counts=True)
    sc2, _ = _make_sc_segsum(N, n_groups, n_pass, with_counts=False)

    xcols = [init_emb[:, p * PCOLS:(p + 1) * PCOLS] for p in range(n_pass)]
    sums1, cnts = sc1(*xcols, src_r, scat_r)

    B = 2000
    NB = N // B
    grid = (NB,)
    f32 = jnp.float32

    spec_x = pl.BlockSpec((B, D), lambda i: (i, 0))
    spec_sp = pl.BlockSpec((NC, n_pass, B, PCOLS), lambda i: (0, 0, i, 0))
    spec_sn = pl.BlockSpec((NC, n_pass, B, PCOLS), lambda i: (0, 0, i + NB, 0))
    spec_cp = pl.BlockSpec((NC, B, PCOLS), lambda i: (0, i, 0))
    spec_cn = pl.BlockSpec((NC, B, PCOLS), lambda i: (0, i + NB, 0))

    def whole(shape):
        nd = len(shape)
        return pl.BlockSpec(shape, lambda i, _n=nd: (0,) * _n)

    z1, *zcols = pl.pallas_call(
        _layer1_body,
        grid=grid,
        in_specs=[spec_x, spec_sp, spec_sn, spec_cp, spec_cn,
                  whole((2 * D, H)), whole((1, H)),
                  whole((2 * D, H)), whole((1, H))],
        out_specs=[pl.BlockSpec((B, D), lambda i: (i, 0))]
        + [pl.BlockSpec((B, PCOLS), lambda i: (i, 0))] * n_pass,
        out_shape=[jax.ShapeDtypeStruct((N, D), f32)]
        + [jax.ShapeDtypeStruct((N, PCOLS), f32)] * n_pass,
    )(init_emb, sums1, sums1, cnts, cnts, Wb1, bb1.reshape(1, H),
      Wu1, bu1.reshape(1, H))

    (sums2,) = sc2(*zcols, src_r, scat_r)

    z, h1p, stats1 = pl.pallas_call(
        _layer2_body,
        grid=grid,
        in_specs=[spec_x, spec_sp, spec_sn, spec_cp, spec_cn,
                  whole((3 * H, H)), whole((1, H)),
                  whole((3 * H, H)), whole((1, H)),
                  whole((D, D)), whole((1, D)),
                  whole((D, D)), whole((1, D))],
        out_specs=[pl.BlockSpec((B, D), lambda i: (i, 0)),
                   pl.BlockSpec((B, D), lambda i: (i, 0)),
                   pl.BlockSpec((2, D), lambda i: (0, 0))],
        out_shape=[jax.ShapeDtypeStruct((N, D), f32),
                   jax.ShapeDtypeStruct((N, D), f32),
                   jax.ShapeDtypeStruct((2, D), f32)],
    )(z1, sums2, sums2, cnts, cnts, Wb2, bb2.reshape(1, H),
      Wu2, bu2.reshape(1, H), Ww, bw.reshape(1, D), M1, c1.reshape(1, D))

    h2p, stats2 = pl.pallas_call(
        functools.partial(_mlp_mid_body, float(N)),
        grid=grid,
        in_specs=[spec_x, whole((2, D)), whole((1, D)), whole((1, D)),
                  whole((D, D)), whole((1, D))],
        out_specs=[pl.BlockSpec((B, D), lambda i: (i, 0)),
                   pl.BlockSpec((2, D), lambda i: (0, 0))],
        out_shape=[jax.ShapeDtypeStruct((N, D), f32),
                   jax.ShapeDtypeStruct((2, D), f32)],
    )(h1p, stats1, g1.reshape(1, D), be1.reshape(1, D), M2, c2.reshape(1, D))

    prob = pl.pallas_call(
        functools.partial(_mlp_out_body, float(N)),
        grid=grid,
        in_specs=[spec_x, whole((2, D)), whole((1, D)), whole((1, D)),
                  whole((1, D)), whole((1, 1))],
        out_specs=pl.BlockSpec((B, 1), lambda i: (i, 0)),
        out_shape=jax.ShapeDtypeStruct((N, 1), f32),
    )(h2p, stats2, g2.reshape(1, D), be2.reshape(1, D),
      M3.reshape(1, D), c3.reshape(1, 1))

    return (z, prob)


# speedup vs baseline: 13.9517x; 1.5326x over previous
import functools

import jax
import jax.numpy as jnp
from jax import lax
from jax.experimental import pallas as pl
from jax.experimental.pallas import tpu as pltpu
from jax.experimental.pallas import tpu_sc as plsc

NC = 2
NS = 16
NW = NC * NS
CHUNK = 128
SUB = 8
NSUB = 5
GC = SUB * NSUB
NBUF = 8
PCOLS = 32
TDT = jnp.bfloat16


def _round_up(a, b):
    return (a + b - 1) // b * b


def _make_sc_segsum(n_nodes, n_groups, n_pass, with_counts):
    R = _round_up(2 * n_nodes + 8, NS * 8)
    stripe = R // NS
    nzfull = stripe // CHUNK
    ztail = stripe - nzfull * CHUNK

    out_type = [jax.ShapeDtypeStruct((NC, n_pass, R, PCOLS), TDT)]
    if with_counts:
        out_type.append(jax.ShapeDtypeStruct((NC, R, PCOLS), TDT))

    mesh = plsc.VectorSubcoreMesh(core_axis_name="c", subcore_axis_name="s")

    scratch = (
        [pltpu.VMEM((GC, CHUNK), jnp.int32),
         pltpu.VMEM((GC, CHUNK), jnp.int32)]
        + [pltpu.VMEM((CHUNK, PCOLS), TDT) for _ in range(NBUF)]
        + [pltpu.VMEM_SHARED((R, PCOLS), TDT),
           pltpu.SemaphoreType.DMA,
           pltpu.SemaphoreType.DMA]
    )

    @functools.partial(pl.kernel, out_type=out_type, mesh=mesh,
                       scratch_types=scratch,
                       compiler_params=pltpu.CompilerParams(
                           use_tc_tiling_on_sc=False))
    def kern(*refs):
        tabs = refs[:n_pass]
        src_hbm, scat_hbm = refs[n_pass], refs[n_pass + 1]
        sums_hbm = refs[n_pass + 2]
        k = n_pass + 3
        cnts_hbm = refs[k] if with_counts else None
        k += 1 if with_counts else 0
        srcb, scatb = refs[k], refs[k + 1]
        rows = refs[k + 2:k + 2 + NBUF]
        acc, gsem, ssem = refs[k + 2 + NBUF:k + 5 + NBUF]

        c = lax.axis_index("c")
        s = lax.axis_index("s")
        wid = c * NS + s
        base = s * stripe

        def _fill(buf, val):
            def f(i, carry):
                buf[i] = jnp.full((PCOLS,), val, TDT)
                return carry
            lax.fori_loop(0, CHUNK, f, 0)

        def _zero_stripe():
            def zf(i, carry):
                pltpu.async_copy(
                    rows[0], acc.at[pl.ds(base + i * CHUNK, CHUNK)], ssem)
                return carry
            lax.fori_loop(0, nzfull, zf, 0)

            def zw(i, carry):
                pltpu.make_async_copy(
                    rows[0], acc.at[pl.ds(base + i * CHUNK, CHUNK)],
                    ssem).wait()
                return carry
            lax.fori_loop(0, nzfull, zw, 0)
            if ztail:
                pltpu.sync_copy(
                    rows[0].at[pl.ds(0, ztail)],
                    acc.at[pl.ds(base + nzfull * CHUNK, ztail)])

        H1 = SUB // 2

        for p in range(n_pass):
            _fill(rows[0], 0.0)
            _zero_stripe()
            plsc.subcore_barrier()

            def _group(g, carry):
                pltpu.sync_copy(src_hbm.at[wid, g], srcb)
                pltpu.sync_copy(scat_hbm.at[wid, g], scatb)

                def _sub(t, carry2):
                    b0 = t * SUB
                    gd = [pltpu.async_copy(tabs[p].at[srcb.at[b0 + j]],
                                           rows[j], gsem)
                          for j in range(SUB)]
                    for j in range(H1):
                        gd[j].wait()
                    sd = [pltpu.async_copy(rows[j], acc.at[scatb.at[b0 + j]],
                                           ssem, add=True)
                          for j in range(H1)]
                    for j in range(H1, SUB):
                        gd[j].wait()
                    for d in sd:
                        d.wait()
                    sd2 = [pltpu.async_copy(rows[j], acc.at[scatb.at[b0 + j]],
                                            ssem, add=True)
                           for j in range(H1, SUB)]
                    for d in sd2:
                        d.wait()
                    return carry2
                return lax.fori_loop(0, NSUB, _sub, carry)
            lax.fori_loop(0, n_groups, _group, 0)
            plsc.subcore_barrier()
            pltpu.sync_copy(acc.at[pl.ds(base, stripe)],
                            sums_hbm.at[c, p, pl.ds(base, stripe)])

        if with_counts:
            _fill(rows[0], 0.0)
            _zero_stripe()
            plsc.subcore_barrier()
            for b in range(NBUF):
                _fill(rows[b], 1.0)

            def _cgroup(g, carry):
                pltpu.sync_copy(scat_hbm.at[wid, g], scatb)

                def _csub(t, carry2):
                    b0 = t * SUB
                    sd = [pltpu.async_copy(rows[j], acc.at[scatb.at[b0 + j]],
                                           ssem, add=True)
                          for j in range(SUB)]
                    for d in sd:
                        d.wait()
                    return carry2
                return lax.fori_loop(0, NSUB, _csub, carry)
            lax.fori_loop(0, n_groups, _cgroup, 0)
            plsc.subcore_barrier()
            pltpu.sync_copy(acc.at[pl.ds(base, stripe)],
                            cnts_hbm.at[c, pl.ds(base, stripe)])

    return kern, R


def _sum_cores_concat(t):
    ts = t[0].astype(jnp.float32) + t[1].astype(jnp.float32)
    return jnp.concatenate([ts[p] for p in range(ts.shape[0])], axis=-1)


def _mean(sums_blk, cnt_blk):
    cnt = (cnt_blk[0].astype(jnp.float32)
           + cnt_blk[1].astype(jnp.float32))[:, 0:1]
    return _sum_cores_concat(sums_blk) / jnp.maximum(cnt, 1.0)


def _dot(a, b):
    return jnp.dot(a, b, preferred_element_type=jnp.float32)


def _layer1_body(xr, spr, snr, cpr, cnr, wb, bb, wu, bu, z1r, *zcr):
    x = xr[...]
    mp = _mean(spr[...], cpr[...])
    mn = _mean(snr[...], cnr[...])
    hb = _dot(mp, wb[0:64]) + _dot(x, wb[64:128]) + bb[...]
    hu = _dot(mn, wu[0:64]) + _dot(x, wu[64:128]) + bu[...]
    z1 = jnp.tanh(jnp.concatenate([hb, hu], axis=1))
    z1r[...] = z1
    for p, r in enumerate(zcr):
        r[...] = z1[:, p * PCOLS:(p + 1) * PCOLS].astype(TDT)


def _layer2_body(z1r, spr, snr, cpr, cnr, wb, bb, wu, bu, ww, bw, m1, c1,
                 zr, h1pr, statsr):
    z1 = z1r[...]
    mp = _mean(spr[...], cpr[...])
    mn = _mean(snr[...], cnr[...])
    hb = (_dot(mp[:, 0:32], wb[0:32]) + _dot(mn[:, 32:64], wb[32:64])
          + _dot(z1[:, 0:32], wb[64:96]) + bb[...])
    hu = (_dot(mp[:, 32:64], wu[0:32]) + _dot(mn[:, 0:32], wu[32:64])
          + _dot(z1[:, 32:64], wu[64:96]) + bu[...])
    z2 = jnp.tanh(jnp.concatenate([hb, hu], axis=1))
    z = jnp.tanh(_dot(z2, ww[...]) + bw[...])
    zr[...] = z
    h1p = _dot(z, m1[...]) + c1[...]
    h1pr[...] = h1p

    @pl.when(pl.program_id(0) == 0)
    def _():
        statsr[...] = jnp.zeros_like(statsr)
    statsr[0:1, :] += jnp.sum(h1p, axis=0, keepdims=True)
    statsr[1:2, :] += jnp.sum(h1p * h1p, axis=0, keepdims=True)


def _bn_relu(h, stats, g, b, n):
    m = stats[0:1, :] / n
    v = stats[1:2, :] / n - m * m
    return jnp.maximum((h - m) * lax.rsqrt(v + 1e-5) * g + b, 0.0)


def _mlp_mid_body(n_nodes, h1pr, statsr, g1, be1, m2, c2, h2pr, stats2r):
    h1 = _bn_relu(h1pr[...], statsr[...], g1[...], be1[...], n_nodes)
    h2p = _dot(h1, m2[...]) + c2[...]
    h2pr[...] = h2p

    @pl.when(pl.program_id(0) == 0)
    def _():
        stats2r[...] = jnp.zeros_like(stats2r)
    stats2r[0:1, :] += jnp.sum(h2p, axis=0, keepdims=True)
    stats2r[1:2, :] += jnp.sum(h2p * h2p, axis=0, keepdims=True)


def _mlp_out_body(n_nodes, h2pr, statsr, g2, be2, m3t, c3, probr):
    h2 = _bn_relu(h2pr[...], statsr[...], g2[...], be2[...], n_nodes)
    logit = jnp.sum(h2 * m3t[...], axis=1, keepdims=True) + c3[...]
    probr[...] = jax.nn.sigmoid(logit)


def kernel(init_emb, edge_index_s, Wb1, bb1, Wu1, bu1, Wb2, bb2, Wu2, bu2,
           Ww, bw, M1, c1, g1, be1, M2, c2, g2, be2, M3, c3):
    N, D = init_emb.shape
    E = edge_index_s.shape[0]
    H = Wb1.shape[1]
    n_pass = D // PCOLS
    n_groups = _round_up(E, NW * GC * CHUNK) // (NW * GC * CHUNK)
    Ep = NW * n_groups * GC * CHUNK

    src = edge_index_s[:, 0].astype(jnp.int32)
    dst = edge_index_s[:, 1].astype(jnp.int32)
    sgn = edge_index_s[:, 2]
    scat = dst + N * (sgn < 0).astype(jnp.int32)
    pad_ar = jnp.arange(Ep - E, dtype=jnp.int32)
    src_r = jnp.concatenate([src, pad_ar % N]).reshape(NW, n_groups, GC, CHUNK)
    scat_r = jnp.concatenate([scat, 2 * N + pad_ar % 88]).reshape(
        NW, n_groups, GC, CHUNK)

    sc1, R = _make_sc_segsum(N, n_groups, n_pass, with_counts=True)
    sc2, _ = _make_sc_segsum(N, n_groups, n_pass, with_counts=False)

    xcols = [init_emb[:, p * PCOLS:(p + 1) * PCOLS].astype(TDT)
             for p in range(n_pass)]
    sums1, cnts = sc1(*xcols, src_r, scat_r)

    B = 2000
    NB = N // B
    grid = (NB,)
    f32 = jnp.float32

    spec_x = pl.BlockSpec((B, D), lambda i: (i, 0))
    spec_sp = pl.BlockSpec((NC, n_pass, B, PCOLS), lambda i: (0, 0, i, 0))
    spec_sn = pl.BlockSpec((NC, n_pass, B, PCOLS), lambda i: (0, 0, i + NB, 0))
    spec_cp = pl.BlockSpec((NC, B, PCOLS), lambda i: (0, i, 0))
    spec_cn = pl.BlockSpec((NC, B, PCOLS), lambda i: (0, i + NB, 0))

    def whole(shape):
        nd = len(shape)
        return pl.BlockSpec(shape, lambda i, _n=nd: (0,) * _n)

    z1, *zcols = pl.pallas_call(
        _layer1_body,
        grid=grid,
        in_specs=[spec_x, spec_sp, spec_sn, spec_cp, spec_cn,
                  whole((2 * D, H)), whole((1, H)),
                  whole((2 * D, H)), whole((1, H))],
        out_specs=[pl.BlockSpec((B, D), lambda i: (i, 0))]
        + [pl.BlockSpec((B, PCOLS), lambda i: (i, 0))] * n_pass,
        out_shape=[jax.ShapeDtypeStruct((N, D), f32)]
        + [jax.ShapeDtypeStruct((N, PCOLS), TDT)] * n_pass,
    )(init_emb, sums1, sums1, cnts, cnts, Wb1, bb1.reshape(1, H),
      Wu1, bu1.reshape(1, H))

    (sums2,) = sc2(*zcols, src_r, scat_r)

    z, h1p, stats1 = pl.pallas_call(
        _layer2_body,
        grid=grid,
        in_specs=[spec_x, spec_sp, spec_sn, spec_cp, spec_cn,
                  whole((3 * H, H)), whole((1, H)),
                  whole((3 * H, H)), whole((1, H)),
                  whole((D, D)), whole((1, D)),
                  whole((D, D)), whole((1, D))],
        out_specs=[pl.BlockSpec((B, D), lambda i: (i, 0)),
                   pl.BlockSpec((B, D), lambda i: (i, 0)),
                   pl.BlockSpec((2, D), lambda i: (0, 0))],
        out_shape=[jax.ShapeDtypeStruct((N, D), f32),
                   jax.ShapeDtypeStruct((N, D), f32),
                   jax.ShapeDtypeStruct((2, D), f32)],
    )(z1, sums2, sums2, cnts, cnts, Wb2, bb2.reshape(1, H),
      Wu2, bu2.reshape(1, H), Ww, bw.reshape(1, D), M1, c1.reshape(1, D))

    h2p, stats2 = pl.pallas_call(
        functools.partial(_mlp_mid_body, float(N)),
        grid=grid,
        in_specs=[spec_x, whole((2, D)), whole((1, D)), whole((1, D)),
                  whole((D, D)), whole((1, D))],
        out_specs=[pl.BlockSpec((B, D), lambda i: (i, 0)),
                   pl.BlockSpec((2, D), lambda i: (0, 0))],
        out_shape=[jax.ShapeDtypeStruct((N, D), f32),
                   jax.ShapeDtypeStruct((2, D), f32)],
    )(h1p, stats1, g1.reshape(1, D), be1.reshape(1, D), M2, c2.reshape(1, D))

    prob = pl.pallas_call(
        functools.partial(_mlp_out_body, float(N)),
        grid=grid,
        in_specs=[spec_x, whole((2, D)), whole((1, D)), whole((1, D)),
                  whole((1, D)), whole((1, 1))],
        out_specs=pl.BlockSpec((B, 1), lambda i: (i, 0)),
        out_shape=jax.ShapeDtypeStruct((N, 1), f32),
    )(h2p, stats2, g2.reshape(1, D), be2.reshape(1, D),
      M3.reshape(1, D), c3.reshape(1, 1))

    return (z, prob)


# speedup vs baseline: 14.2669x; 1.0226x over previous
import functools

import jax
import jax.numpy as jnp
from jax import lax
from jax.experimental import pallas as pl
from jax.experimental.pallas import tpu as pltpu
from jax.experimental.pallas import tpu_sc as plsc

NC = 2
NS = 16
NW = NC * NS
CHUNK = 128
SUB = 8
NSUB = 5
GC = SUB * NSUB
NBUF = 8
PCOLS = 32
TDT = jnp.bfloat16


def _round_up(a, b):
    return (a + b - 1) // b * b


def _make_sc_segsum(n_nodes, n_groups, n_pass, with_counts):
    R = _round_up(2 * n_nodes + 8, NS * 8)
    stripe = R // NS
    nzfull = stripe // CHUNK
    ztail = stripe - nzfull * CHUNK

    out_type = [jax.ShapeDtypeStruct((NC, n_pass, R, PCOLS), TDT)]
    if with_counts:
        out_type.append(jax.ShapeDtypeStruct((NC, R, PCOLS), TDT))

    mesh = plsc.VectorSubcoreMesh(core_axis_name="c", subcore_axis_name="s")

    scratch = (
        [pltpu.VMEM((GC, CHUNK), jnp.int32),
         pltpu.VMEM((GC, CHUNK), jnp.int32)]
        + [pltpu.VMEM((CHUNK, PCOLS), TDT) for _ in range(NBUF)]
        + [pltpu.VMEM_SHARED((R, PCOLS), TDT),
           pltpu.SemaphoreType.DMA,
           pltpu.SemaphoreType.DMA]
    )

    @functools.partial(pl.kernel, out_type=out_type, mesh=mesh,
                       scratch_types=scratch,
                       compiler_params=pltpu.CompilerParams(
                           use_tc_tiling_on_sc=False))
    def kern(*refs):
        tabs = refs[:n_pass]
        src_hbm, scat_hbm = refs[n_pass], refs[n_pass + 1]
        sums_hbm = refs[n_pass + 2]
        k = n_pass + 3
        cnts_hbm = refs[k] if with_counts else None
        k += 1 if with_counts else 0
        srcb, scatb = refs[k], refs[k + 1]
        rows = refs[k + 2:k + 2 + NBUF]
        acc, gsem, ssem = refs[k + 2 + NBUF:k + 5 + NBUF]

        c = lax.axis_index("c")
        s = lax.axis_index("s")
        wid = c * NS + s
        base = s * stripe

        def _fill(buf, val):
            def f(i, carry):
                buf[i] = jnp.full((PCOLS,), val, TDT)
                return carry
            lax.fori_loop(0, CHUNK, f, 0)

        def _zero_stripe():
            def zf(i, carry):
                pltpu.async_copy(
                    rows[0], acc.at[pl.ds(base + i * CHUNK, CHUNK)], ssem)
                return carry
            lax.fori_loop(0, nzfull, zf, 0)

            def zw(i, carry):
                pltpu.make_async_copy(
                    rows[0], acc.at[pl.ds(base + i * CHUNK, CHUNK)],
                    ssem).wait()
                return carry
            lax.fori_loop(0, nzfull, zw, 0)
            if ztail:
                pltpu.sync_copy(
                    rows[0].at[pl.ds(0, ztail)],
                    acc.at[pl.ds(base + nzfull * CHUNK, ztail)])

        H1 = SUB // 2

        for p in range(n_pass):
            _fill(rows[0], 0.0)
            _zero_stripe()
            plsc.subcore_barrier()

            def _drain4():
                for _ in range(H1):
                    pltpu.make_async_copy(rows[H1],
                                          acc.at[scatb.at[0]], ssem).wait()

            def _group(g, carry):
                @pl.when(g > 0)
                def _():
                    _drain4()
                pltpu.sync_copy(src_hbm.at[wid, g], srcb)
                pltpu.sync_copy(scat_hbm.at[wid, g], scatb)

                def _sub(t, carry2):
                    b0 = t * SUB
                    gd = [pltpu.async_copy(tabs[p].at[srcb.at[b0 + j]],
                                           rows[j], gsem)
                          for j in range(H1)]
                    @pl.when(t > 0)
                    def _():
                        _drain4()
                    gd += [pltpu.async_copy(tabs[p].at[srcb.at[b0 + j]],
                                            rows[j], gsem)
                           for j in range(H1, SUB)]
                    for j in range(H1):
                        gd[j].wait()
                    sd = [pltpu.async_copy(rows[j], acc.at[scatb.at[b0 + j]],
                                           ssem, add=True)
                          for j in range(H1)]
                    for j in range(H1, SUB):
                        gd[j].wait()
                    for d in sd:
                        d.wait()
                    for j in range(H1, SUB):
                        pltpu.async_copy(rows[j], acc.at[scatb.at[b0 + j]],
                                         ssem, add=True)
                    return carry2
                return lax.fori_loop(0, NSUB, _sub, carry)
            lax.fori_loop(0, n_groups, _group, 0)
            _drain4()
            plsc.subcore_barrier()
            pltpu.sync_copy(acc.at[pl.ds(base, stripe)],
                            sums_hbm.at[c, p, pl.ds(base, stripe)])

        if with_counts:
            _fill(rows[0], 0.0)
            _zero_stripe()
            plsc.subcore_barrier()
            for b in range(NBUF):
                _fill(rows[b], 1.0)

            def _cgroup(g, carry):
                pltpu.sync_copy(scat_hbm.at[wid, g], scatb)

                def _csub(t, carry2):
                    b0 = t * SUB
                    sd = [pltpu.async_copy(rows[j], acc.at[scatb.at[b0 + j]],
                                           ssem, add=True)
                          for j in range(SUB)]
                    for d in sd:
                        d.wait()
                    return carry2
                return lax.fori_loop(0, NSUB, _csub, carry)
            lax.fori_loop(0, n_groups, _cgroup, 0)
            plsc.subcore_barrier()
            pltpu.sync_copy(acc.at[pl.ds(base, stripe)],
                            cnts_hbm.at[c, pl.ds(base, stripe)])

    return kern, R


def _sum_cores_concat(t):
    ts = t[0].astype(jnp.float32) + t[1].astype(jnp.float32)
    return jnp.concatenate([ts[p] for p in range(ts.shape[0])], axis=-1)


def _mean(sums_blk, cnt_blk):
    cnt = (cnt_blk[0].astype(jnp.float32)
           + cnt_blk[1].astype(jnp.float32))[:, 0:1]
    return _sum_cores_concat(sums_blk) / jnp.maximum(cnt, 1.0)


def _dot(a, b):
    return jnp.dot(a, b, preferred_element_type=jnp.float32)


def _layer1_body(xr, spr, snr, cpr, cnr, wb, bb, wu, bu, z1r, *zcr):
    x = xr[...]
    mp = _mean(spr[...], cpr[...])
    mn = _mean(snr[...], cnr[...])
    hb = _dot(mp, wb[0:64]) + _dot(x, wb[64:128]) + bb[...]
    hu = _dot(mn, wu[0:64]) + _dot(x, wu[64:128]) + bu[...]
    z1 = jnp.tanh(jnp.concatenate([hb, hu], axis=1))
    z1r[...] = z1
    for p, r in enumerate(zcr):
        r[...] = z1[:, p * PCOLS:(p + 1) * PCOLS].astype(TDT)


def _layer2_body(z1r, spr, snr, cpr, cnr, wb, bb, wu, bu, ww, bw, m1, c1,
                 zr, h1pr, statsr):
    z1 = z1r[...]
    mp = _mean(spr[...], cpr[...])
    mn = _mean(snr[...], cnr[...])
    hb = (_dot(mp[:, 0:32], wb[0:32]) + _dot(mn[:, 32:64], wb[32:64])
          + _dot(z1[:, 0:32], wb[64:96]) + bb[...])
    hu = (_dot(mp[:, 32:64], wu[0:32]) + _dot(mn[:, 0:32], wu[32:64])
          + _dot(z1[:, 32:64], wu[64:96]) + bu[...])
    z2 = jnp.tanh(jnp.concatenate([hb, hu], axis=1))
    z = jnp.tanh(_dot(z2, ww[...]) + bw[...])
    zr[...] = z
    h1p = _dot(z, m1[...]) + c1[...]
    h1pr[...] = h1p

    @pl.when(pl.program_id(0) == 0)
    def _():
        statsr[...] = jnp.zeros_like(statsr)
    statsr[0:1, :] += jnp.sum(h1p, axis=0, keepdims=True)
    statsr[1:2, :] += jnp.sum(h1p * h1p, axis=0, keepdims=True)


def _bn_relu(h, stats, g, b, n):
    m = stats[0:1, :] / n
    v = stats[1:2, :] / n - m * m
    return jnp.maximum((h - m) * lax.rsqrt(v + 1e-5) * g + b, 0.0)


def _mlp_mid_body(n_nodes, h1pr, statsr, g1, be1, m2, c2, h2pr, stats2r):
    h1 = _bn_relu(h1pr[...], statsr[...], g1[...], be1[...], n_nodes)
    h2p = _dot(h1, m2[...]) + c2[...]
    h2pr[...] = h2p

    @pl.when(pl.program_id(0) == 0)
    def _():
        stats2r[...] = jnp.zeros_like(stats2r)
    stats2r[0:1, :] += jnp.sum(h2p, axis=0, keepdims=True)
    stats2r[1:2, :] += jnp.sum(h2p * h2p, axis=0, keepdims=True)


def _mlp_out_body(n_nodes, h2pr, statsr, g2, be2, m3t, c3, probr):
    h2 = _bn_relu(h2pr[...], statsr[...], g2[...], be2[...], n_nodes)
    logit = jnp.sum(h2 * m3t[...], axis=1, keepdims=True) + c3[...]
    probr[...] = jax.nn.sigmoid(logit)


def kernel(init_emb, edge_index_s, Wb1, bb1, Wu1, bu1, Wb2, bb2, Wu2, bu2,
           Ww, bw, M1, c1, g1, be1, M2, c2, g2, be2, M3, c3):
    N, D = init_emb.shape
    E = edge_index_s.shape[0]
    H = Wb1.shape[1]
    n_pass = D // PCOLS
    n_groups = _round_up(E, NW * GC * CHUNK) // (NW * GC * CHUNK)
    Ep = NW * n_groups * GC * CHUNK

    src = edge_index_s[:, 0].astype(jnp.int32)
    dst = edge_index_s[:, 1].astype(jnp.int32)
    sgn = edge_index_s[:, 2]
    scat = dst + N * (sgn < 0).astype(jnp.int32)
    pad_ar = jnp.arange(Ep - E, dtype=jnp.int32)
    src_r = jnp.concatenate([src, pad_ar % N]).reshape(NW, n_groups, GC, CHUNK)
    scat_r = jnp.concatenate([scat, 2 * N + pad_ar % 88]).reshape(
        NW, n_groups, GC, CHUNK)

    sc1, R = _make_sc_segsum(N, n_groups, n_pass, with_counts=True)
    sc2, _ = _make_sc_segsum(N, n_groups, n_pass, with_counts=False)

    xcols = [init_emb[:, p * PCOLS:(p + 1) * PCOLS].astype(TDT)
             for p in range(n_pass)]
    sums1, cnts = sc1(*xcols, src_r, scat_r)

    B = 2000
    NB = N // B
    grid = (NB,)
    f32 = jnp.float32

    spec_x = pl.BlockSpec((B, D), lambda i: (i, 0))
    spec_sp = pl.BlockSpec((NC, n_pass, B, PCOLS), lambda i: (0, 0, i, 0))
    spec_sn = pl.BlockSpec((NC, n_pass, B, PCOLS), lambda i: (0, 0, i + NB, 0))
    spec_cp = pl.BlockSpec((NC, B, PCOLS), lambda i: (0, i, 0))
    spec_cn = pl.BlockSpec((NC, B, PCOLS), lambda i: (0, i + NB, 0))

    def whole(shape):
        nd = len(shape)
        return pl.BlockSpec(shape, lambda i, _n=nd: (0,) * _n)

    z1, *zcols = pl.pallas_call(
        _layer1_body,
        grid=grid,
        in_specs=[spec_x, spec_sp, spec_sn, spec_cp, spec_cn,
                  whole((2 * D, H)), whole((1, H)),
                  whole((2 * D, H)), whole((1, H))],
        out_specs=[pl.BlockSpec((B, D), lambda i: (i, 0))]
        + [pl.BlockSpec((B, PCOLS), lambda i: (i, 0))] * n_pass,
        out_shape=[jax.ShapeDtypeStruct((N, D), f32)]
        + [jax.ShapeDtypeStruct((N, PCOLS), TDT)] * n_pass,
    )(init_emb, sums1, sums1, cnts, cnts, Wb1, bb1.reshape(1, H),
      Wu1, bu1.reshape(1, H))

    (sums2,) = sc2(*zcols, src_r, scat_r)

    z, h1p, stats1 = pl.pallas_call(
        _layer2_body,
        grid=grid,
        in_specs=[spec_x, spec_sp, spec_sn, spec_cp, spec_cn,
                  whole((3 * H, H)), whole((1, H)),
                  whole((3 * H, H)), whole((1, H)),
                  whole((D, D)), whole((1, D)),
                  whole((D, D)), whole((1, D))],
        out_specs=[pl.BlockSpec((B, D), lambda i: (i, 0)),
                   pl.BlockSpec((B, D), lambda i: (i, 0)),
                   pl.BlockSpec((2, D), lambda i: (0, 0))],
        out_shape=[jax.ShapeDtypeStruct((N, D), f32),
                   jax.ShapeDtypeStruct((N, D), f32),
                   jax.ShapeDtypeStruct((2, D), f32)],
    )(z1, sums2, sums2, cnts, cnts, Wb2, bb2.reshape(1, H),
      Wu2, bu2.reshape(1, H), Ww, bw.reshape(1, D), M1, c1.reshape(1, D))

    h2p, stats2 = pl.pallas_call(
        functools.partial(_mlp_mid_body, float(N)),
        grid=grid,
        in_specs=[spec_x, whole((2, D)), whole((1, D)), whole((1, D)),
                  whole((D, D)), whole((1, D))],
        out_specs=[pl.BlockSpec((B, D), lambda i: (i, 0)),
                   pl.BlockSpec((2, D), lambda i: (0, 0))],
        out_shape=[jax.ShapeDtypeStruct((N, D), f32),
                   jax.ShapeDtypeStruct((2, D), f32)],
    )(h1p, stats1, g1.reshape(1, D), be1.reshape(1, D), M2, c2.reshape(1, D))

    prob = pl.pallas_call(
        functools.partial(_mlp_out_body, float(N)),
        grid=grid,
        in_specs=[spec_x, whole((2, D)), whole((1, D)), whole((1, D)),
                  whole((1, D)), whole((1, 1))],
        out_specs=pl.BlockSpec((B, 1), lambda i: (i, 0)),
        out_shape=jax.ShapeDtypeStruct((N, 1), f32),
    )(h2p, stats2, g2.reshape(1, D), be2.reshape(1, D),
      M3.reshape(1, D), c3.reshape(1, 1))

    return (z, prob)
